# Initial kernel scaffold; baseline (speedup 1.0000x reference)
import functools

import jax
import jax.numpy as jnp
from jax import lax
from jax.experimental import pallas as pl
from jax.experimental.pallas import tpu as pltpu
from jax.experimental.pallas import tpu_sc as plsc

N = 320000
D = 128
S = 10000
SP = 10240
NC = 2
NS = 16
NW = NC * NS
RPT = N // NW
CHUNK = 80
NCHUNK = RPT // CHUNK
L = 16


def _sc_body(feat_hbm, pidx_hbm, pidx2d_hbm, psum_hbm, pcnt_hbm,
             rows_v, idx2d_v, ids_v, firstp, lastp, cntl, identr, zbuf,
             acc_sh, cnt_sh):
    c = lax.axis_index("c")
    s = lax.axis_index("s")
    wid = c * NS + s
    base = wid * RPT

    zi = jnp.zeros((L,), jnp.int32)
    zf = jnp.zeros((L,), jnp.float32)
    iota = lax.iota(jnp.int32, L)

    @pl.loop(0, (125 * D) // L)
    def _(m):
        zbuf[m // 8, pl.ds((m % 8) * L, L)] = zf

    @pl.loop(0, SP // L)
    def _(k):
        firstp[pl.ds(k * L, L)] = zi
        lastp[pl.ds(k * L, L)] = zi

    @pl.loop(0, CHUNK // L)
    def _(j):
        identr[0, pl.ds(j * L, L)] = iota + j * L

    @pl.loop(0, 5)
    def _(i):
        pltpu.sync_copy(zbuf, acc_sh.at[pl.ds(s * 625 + i * 125, 125)])

    pltpu.sync_copy(zbuf.at[pl.ds(0, 5)], cnt_sh.at[pl.ds(s * 5, 5)])

    pltpu.sync_copy(pidx_hbm.at[pl.ds(base, RPT)], ids_v.at[pl.ds(L, RPT)])
    pltpu.sync_copy(pidx2d_hbm.at[pl.ds(wid * NCHUNK, NCHUNK)], idx2d_v)
    neg1 = jnp.full((L,), -1, jnp.int32)
    ids_v[pl.ds(0, L)] = neg1
    ids_v[pl.ds(L + RPT, L)] = neg1

    plsc.subcore_barrier()

    @pl.loop(0, NCHUNK)
    def _(j):
        pltpu.sync_copy(feat_hbm.at[pl.ds(base + j * CHUNK, CHUNK)], rows_v)
        pltpu.sync_copy(rows_v, acc_sh.at[idx2d_v.at[j]], add=True)

    @pl.loop(0, RPT // L)
    def _(k):
        i = k * L
        v = ids_v[pl.ds(i + L, L)]
        vl = ids_v[pl.ds(i + L - 1, L)]
        vr = ids_v[pl.ds(i + L + 1, L)]
        pos = iota + i
        plsc.store_scatter(firstp, [v], pos, mask=v != vl)
        plsc.store_scatter(lastp, [v], pos + 1, mask=v != vr)

    @pl.loop(0, SP // L)
    def _(k):
        cv = (lastp[pl.ds(k * L, L)] - firstp[pl.ds(k * L, L)]).astype(
            jnp.float32)
        cntl[k // 8, pl.ds((k % 8) * L, L)] = cv

    pltpu.sync_copy(cntl, cnt_sh.at[identr.at[0]], add=True)

    plsc.subcore_barrier()

    @pl.loop(0, 5)
    def _(i):
        r0 = s * 625 + i * 125
        pltpu.sync_copy(acc_sh.at[pl.ds(r0, 125)],
                        psum_hbm.at[c, pl.ds(r0, 125)])

    pltpu.sync_copy(cnt_sh.at[pl.ds(s * 5, 5)], pcnt_hbm.at[c, pl.ds(s * 5, 5)])


def _sc_segment_sum(feature, point_idx, pidx2d):
    mesh = plsc.VectorSubcoreMesh(core_axis_name="c", subcore_axis_name="s")
    kern = pl.kernel(
        _sc_body,
        out_type=(
            jax.ShapeDtypeStruct((NC, S, D), jnp.float32),
            jax.ShapeDtypeStruct((NC, CHUNK, D), jnp.float32),
        ),
        mesh=mesh,
        scratch_types=[
            pltpu.VMEM((CHUNK, D), jnp.float32),
            pltpu.VMEM((NCHUNK, CHUNK), jnp.int32),
            pltpu.VMEM((RPT + 2 * L,), jnp.int32),
            pltpu.VMEM((SP,), jnp.int32),
            pltpu.VMEM((SP,), jnp.int32),
            pltpu.VMEM((CHUNK, D), jnp.float32),
            pltpu.VMEM((1, CHUNK), jnp.int32),
            pltpu.VMEM((125, D), jnp.float32),
            pltpu.VMEM_SHARED((S, D), jnp.float32),
            pltpu.VMEM_SHARED((CHUNK, D), jnp.float32),
        ],
    )
    return kern(feature, point_idx, pidx2d)


def _tc_body(pa, pb, c0, c1, o):
    tot = c0[0, 0, :] + c1[0, 0, :]
    inv = 1.0 / jnp.maximum(tot, 1.0)
    invc = jnp.reshape(inv, (D, 1))
    o[...] = jax.nn.sigmoid((pa[...] + pb[...]) * invc)


_TC_GRID = (S + D - 1) // D


_tc_finalize = pl.pallas_call(
    _tc_body,
    grid=(_TC_GRID,),
    in_specs=[
        pl.BlockSpec((D, D), lambda i: (i, 0)),
        pl.BlockSpec((D, D), lambda i: (i, 0)),
        pl.BlockSpec((1, 1, D), lambda i: (i, 0, 0)),
        pl.BlockSpec((1, 1, D), lambda i: (i, 0, 0)),
    ],
    out_specs=pl.BlockSpec((D, D), lambda i: (i, 0)),
    out_shape=jax.ShapeDtypeStruct((S, D), jnp.float32),
)


def kernel(feature, point_idx):
    pidx2d = point_idx.reshape(N // CHUNK, CHUNK)
    psum, pcnt = _sc_segment_sum(feature, point_idx, pidx2d)
    c0 = pcnt[0].reshape(SP // D, 1, D)
    c1 = pcnt[1].reshape(SP // D, 1, D)
    return _tc_finalize(psum[0], psum[1], c0, c1)


# speedup vs baseline: 3.8001x; 3.8001x over previous
import dataclasses
import functools

import jax
import jax.numpy as jnp
from jax import lax
from jax.experimental import pallas as pl
from jax.experimental.pallas import tpu as pltpu
from jax.experimental.pallas import tpu_sc as plsc

N = 320000
D = 128
S = 10000
SP = 10240
NC = 2
NS = 16
NW = NC * NS
RPT = N // NW
CHUNK = 80
NCHUNK = RPT // CHUNK
L = 16
SW = 5120


def _sc_body(feat_hbm, pidx_hbm, pidx2d_hbm, psum_hbm, pcnt_hbm,
             rows_v, idx2d_v, ids_v, firstp, lastp, cntl, identr, tidx, zbuf,
             acc_sh, cnt_sh):
    c = lax.axis_index("c")
    s = lax.axis_index("s")
    wid = c * NS + s
    base = wid * RPT

    zi = jnp.zeros((L,), jnp.int32)
    zf = jnp.zeros((L,), jnp.float32)
    iota = lax.iota(jnp.int32, L)

    @pl.loop(0, (80 * D) // L)
    def _(m):
        zbuf[m // 8, pl.ds((m % 8) * L, L)] = zf

    @pl.loop(0, SP // L)
    def _(k):
        firstp[pl.ds(k * L, L)] = zi
        lastp[pl.ds(k * L, L)] = zi

    @pl.loop(0, CHUNK // L)
    def _(j):
        identr[0, pl.ds(j * L, L)] = iota + j * L

    @pl.when(s < 10)
    def _():
        pltpu.sync_copy(zbuf.at[pl.ds(0, 8)], cnt_sh.at[pl.ds(s * 8, 8)])

    pltpu.sync_copy(pidx_hbm.at[pl.ds(base, RPT)], ids_v.at[pl.ds(L, RPT)])
    pltpu.sync_copy(pidx2d_hbm.at[wid], idx2d_v)
    neg1 = jnp.full((L,), -1, jnp.int32)
    ids_v[pl.ds(0, L)] = neg1
    ids_v[pl.ds(L + RPT, L)] = neg1

    for w in range(2):
        lo = w * SW
        @pl.loop(0, 4)
        def _(i):
            pltpu.sync_copy(zbuf, acc_sh.at[pl.ds(s * 320 + i * 80, 80)])

        @pl.when(s == 0)
        def _():
            pltpu.sync_copy(zbuf.at[pl.ds(0, 8)], acc_sh.at[pl.ds(SW, 8)])

        plsc.subcore_barrier()

        @pl.loop(0, NCHUNK)
        def _(j):
            v0 = idx2d_v[j, pl.ds(0, L)]
            v4 = idx2d_v[j, pl.ds(CHUNK - L, L)]
            cmin = jnp.min(v0)
            cmax = jnp.max(v4)

            @pl.when(jnp.logical_and(cmin < lo + SW, cmax >= lo))
            def _():
                @pl.loop(0, CHUNK // L)
                def _(q):
                    vv = idx2d_v[j, pl.ds(q * L, L)]
                    ok = jnp.logical_and(vv >= lo, vv < lo + SW)
                    tidx[0, pl.ds(q * L, L)] = jnp.where(ok, vv - lo, SW)

                pltpu.sync_copy(feat_hbm.at[pl.ds(base + j * CHUNK, CHUNK)],
                                rows_v)
                pltpu.sync_copy(rows_v, acc_sh.at[tidx.at[0]], add=True)

        plsc.subcore_barrier()

        @pl.loop(0, 4)
        def _(i):
            r0 = s * 320 + i * 80
            pltpu.sync_copy(acc_sh.at[pl.ds(r0, 80)],
                            psum_hbm.at[c, pl.ds(lo + r0, 80)])

    @pl.loop(0, RPT // L)
    def _(k):
        i = k * L
        v = ids_v[pl.ds(i + L, L)]
        vl = ids_v[pl.ds(i + L - 1, L)]
        vr = ids_v[pl.ds(i + L + 1, L)]
        pos = iota + i
        plsc.store_scatter(firstp, [v], pos, mask=v != vl)
        plsc.store_scatter(lastp, [v], pos + 1, mask=v != vr)

    @pl.loop(0, SP // L)
    def _(k):
        cv = (lastp[pl.ds(k * L, L)] - firstp[pl.ds(k * L, L)]).astype(
            jnp.float32)
        cntl[k // 8, pl.ds((k % 8) * L, L)] = cv

    pltpu.sync_copy(cntl, cnt_sh.at[identr.at[0]], add=True)

    plsc.subcore_barrier()

    @pl.when(s < 10)
    def _():
        pltpu.sync_copy(cnt_sh.at[pl.ds(s * 8, 8)],
                        pcnt_hbm.at[c, pl.ds(s * 8, 8)])


def _sc_segment_sum(feature, point_idx, pidx2d):
    mesh = plsc.VectorSubcoreMesh(core_axis_name="c", subcore_axis_name="s")
    cp = pltpu.CompilerParams()
    if "needs_layout_passes" in pltpu.CompilerParams.__dataclass_fields__:
        cp = dataclasses.replace(cp, needs_layout_passes=False)
    kern = pl.kernel(
        _sc_body,
        compiler_params=cp,
        out_type=(
            jax.ShapeDtypeStruct((NC, 2 * SW, D), jnp.float32),
            jax.ShapeDtypeStruct((NC, CHUNK, D), jnp.float32),
        ),
        mesh=mesh,
        scratch_types=[
            pltpu.VMEM((CHUNK, D), jnp.float32),
            pltpu.VMEM((NCHUNK, CHUNK), jnp.int32),
            pltpu.VMEM((RPT + 2 * L,), jnp.int32),
            pltpu.VMEM((SP,), jnp.int32),
            pltpu.VMEM((SP,), jnp.int32),
            pltpu.VMEM((CHUNK, D), jnp.float32),
            pltpu.VMEM((1, CHUNK), jnp.int32),
            pltpu.VMEM((1, CHUNK), jnp.int32),
            pltpu.VMEM((80, D), jnp.float32),
            pltpu.VMEM_SHARED((SW + 8, D), jnp.float32),
            pltpu.VMEM_SHARED((CHUNK, D), jnp.float32),
        ],
    )
    return kern(feature, point_idx, pidx2d)


def _tc_body(pa, pb, c0, c1, o):
    tot = c0[0, 0, :] + c1[0, 0, :]
    inv = 1.0 / jnp.maximum(tot, 1.0)
    invc = jnp.reshape(inv, (D, 1))
    o[...] = jax.nn.sigmoid((pa[...] + pb[...]) * invc)


_TC_GRID = (S + D - 1) // D


_tc_finalize = pl.pallas_call(
    _tc_body,
    grid=(_TC_GRID,),
    in_specs=[
        pl.BlockSpec((D, D), lambda i: (i, 0)),
        pl.BlockSpec((D, D), lambda i: (i, 0)),
        pl.BlockSpec((1, 1, D), lambda i: (i, 0, 0)),
        pl.BlockSpec((1, 1, D), lambda i: (i, 0, 0)),
    ],
    out_specs=pl.BlockSpec((D, D), lambda i: (i, 0)),
    out_shape=jax.ShapeDtypeStruct((S, D), jnp.float32),
)


def kernel(feature, point_idx):
    pidx2d = point_idx.reshape(NW, NCHUNK, CHUNK)
    psum, pcnt = _sc_segment_sum(feature, point_idx, pidx2d)
    c0 = pcnt[0].reshape(SP // D, 1, D)
    c1 = pcnt[1].reshape(SP // D, 1, D)
    return _tc_finalize(psum[0], psum[1], c0, c1)


# speedup vs baseline: 5.0997x; 1.3420x over previous
import dataclasses

import jax
import jax.numpy as jnp
from jax import lax
from jax.experimental import pallas as pl
from jax.experimental.pallas import tpu as pltpu
from jax.experimental.pallas import tpu_sc as plsc

N = 320000
D = 128
S = 10000
SP = 10240
NC = 2
NS = 16
NW = NC * NS
RPT = N // NW
CHUNK = 80
NCHUNK = RPT // CHUNK
L = 16
SW = 3840
NWIN = 3
NB = 4


def _sc_body(feat_hbm, pidx_hbm, pidx2d_hbm, psum_hbm, pcnt_hbm,
             rows4, idx2d_v, ids_v, firstp, lastp, cntl, identr, tidx,
             acts, gs0, gs1, gs2, gs3, ss0, ss1, ss2, ss3,
             acc_sh, cnt_sh):
    gsems = (gs0, gs1, gs2, gs3)
    ssems = (ss0, ss1, ss2, ss3)
    c = lax.axis_index("c")
    s = lax.axis_index("s")
    wid = c * NS + s
    base = wid * RPT

    zi = jnp.zeros((L,), jnp.int32)
    zf = jnp.zeros((L,), jnp.float32)
    iota = lax.iota(jnp.int32, L)

    @pl.loop(0, (CHUNK * D) // L)
    def _(m):
        rows4[0, m // 8, pl.ds((m % 8) * L, L)] = zf

    @pl.loop(0, SP // L)
    def _(k):
        firstp[pl.ds(k * L, L)] = zi
        lastp[pl.ds(k * L, L)] = zi

    @pl.loop(0, CHUNK // L)
    def _(j):
        identr[0, pl.ds(j * L, L)] = iota + j * L

    @pl.when(s < 10)
    def _():
        pltpu.sync_copy(rows4.at[0, pl.ds(0, 8)], cnt_sh.at[pl.ds(s * 8, 8)])

    pltpu.sync_copy(pidx_hbm.at[pl.ds(base, RPT)], ids_v.at[pl.ds(L, RPT)])
    pltpu.sync_copy(pidx2d_hbm.at[wid], idx2d_v)
    neg1 = jnp.full((L,), -1, jnp.int32)
    ids_v[pl.ds(0, L)] = neg1
    ids_v[pl.ds(L + RPT, L)] = neg1

    @pl.loop(0, NCHUNK)
    def _(j):
        v0 = idx2d_v[j, pl.ds(0, L)]
        v4 = idx2d_v[j, pl.ds(CHUNK - L, L)]
        cmin = jnp.min(v0)
        cmax = jnp.max(v4)
        for w in range(NWIN):
            acts[w, j] = jnp.logical_and(
                cmin < (w + 1) * SW, cmax >= w * SW).astype(jnp.int32)

    def _transform(j, b, lo):
        @pl.loop(0, CHUNK // L)
        def _(q):
            vv = idx2d_v[j, pl.ds(q * L, L)]
            ok = jnp.logical_and(vv >= lo, vv < lo + SW)
            tidx[b, pl.ds(q * L, L)] = jnp.where(ok, vv - lo, SW)

    def _issue_gather(j, b):
        pltpu.async_copy(feat_hbm.at[pl.ds(base + j * CHUNK, CHUNK)],
                         rows4.at[b], gsems[b])

    def _wait_gather(b):
        pltpu.make_async_copy(feat_hbm.at[pl.ds(base, CHUNK)],
                              rows4.at[b], gsems[b]).wait()

    def _issue_scatter(b):
        pltpu.async_copy(rows4.at[b], acc_sh.at[tidx.at[b]], ssems[b],
                         add=True)

    def _wait_scatter(b):
        pltpu.make_async_copy(rows4.at[b], acc_sh.at[tidx.at[b]],
                              ssems[b]).wait()

    for w in range(NWIN):
        lo = w * SW

        if w > 0:
            @pl.loop(0, (CHUNK * D) // L)
            def _(m):
                rows4[0, m // 8, pl.ds((m % 8) * L, L)] = zf

        @pl.loop(0, 3)
        def _(i):
            pltpu.sync_copy(rows4.at[0],
                            acc_sh.at[pl.ds(s * 240 + i * CHUNK, CHUNK)])

        @pl.when(s == 0)
        def _():
            pltpu.sync_copy(rows4.at[0, pl.ds(0, 8)], acc_sh.at[pl.ds(SW, 8)])

        plsc.subcore_barrier()

        for b in range(NB):
            @pl.when(acts[w, b] == 1)
            def _():
                _issue_gather(b, b)

        @pl.loop(0, (NCHUNK - 1) // NB)
        def _(g):
            for b in range(NB):
                j = NB * g + b

                @pl.when(acts[w, j] == 1)
                def _():
                    _wait_gather(b)
                    _transform(j, b, lo)
                    _issue_scatter(b)

            for b in range(NB):
                j = NB * g + b
                jn = j + NB

                @pl.when(acts[w, j] == 1)
                def _():
                    _wait_scatter(b)

                @pl.when(jn < NCHUNK)
                def _():
                    @pl.when(acts[w, jn] == 1)
                    def _():
                        _issue_gather(jn, b)

        @pl.when(acts[w, NCHUNK - 1] == 1)
        def _():
            _wait_gather(0)
            _transform(NCHUNK - 1, 0, lo)
            _issue_scatter(0)
            _wait_scatter(0)

        plsc.subcore_barrier()

        @pl.loop(0, 3)
        def _(i):
            r0 = s * 240 + i * CHUNK
            pltpu.sync_copy(acc_sh.at[pl.ds(r0, CHUNK)],
                            psum_hbm.at[c, pl.ds(lo + r0, CHUNK)])

    @pl.loop(0, RPT // L)
    def _(k):
        i = k * L
        v = ids_v[pl.ds(i + L, L)]
        vl = ids_v[pl.ds(i + L - 1, L)]
        vr = ids_v[pl.ds(i + L + 1, L)]
        pos = iota + i
        plsc.store_scatter(firstp, [v], pos, mask=v != vl)
        plsc.store_scatter(lastp, [v], pos + 1, mask=v != vr)

    @pl.loop(0, SP // L)
    def _(k):
        cv = (lastp[pl.ds(k * L, L)] - firstp[pl.ds(k * L, L)]).astype(
            jnp.float32)
        cntl[k // 8, pl.ds((k % 8) * L, L)] = cv

    pltpu.sync_copy(cntl, cnt_sh.at[identr.at[0]], add=True)

    plsc.subcore_barrier()

    @pl.when(s < 10)
    def _():
        pltpu.sync_copy(cnt_sh.at[pl.ds(s * 8, 8)],
                        pcnt_hbm.at[c, pl.ds(s * 8, 8)])


def _sc_segment_sum(feature, point_idx, pidx2d):
    mesh = plsc.VectorSubcoreMesh(core_axis_name="c", subcore_axis_name="s")
    cp = pltpu.CompilerParams()
    if "needs_layout_passes" in pltpu.CompilerParams.__dataclass_fields__:
        cp = dataclasses.replace(cp, needs_layout_passes=False)
    kern = pl.kernel(
        _sc_body,
        compiler_params=cp,
        out_type=(
            jax.ShapeDtypeStruct((NC, NWIN * SW, D), jnp.float32),
            jax.ShapeDtypeStruct((NC, CHUNK, D), jnp.float32),
        ),
        mesh=mesh,
        scratch_types=[
            pltpu.VMEM((NB, CHUNK, D), jnp.float32),
            pltpu.VMEM((NCHUNK, CHUNK), jnp.int32),
            pltpu.VMEM((RPT + 2 * L,), jnp.int32),
            pltpu.VMEM((SP,), jnp.int32),
            pltpu.VMEM((SP,), jnp.int32),
            pltpu.VMEM((CHUNK, D), jnp.float32),
            pltpu.VMEM((1, CHUNK), jnp.int32),
            pltpu.VMEM((NB, CHUNK), jnp.int32),
            pltpu.SMEM((NWIN, NCHUNK), jnp.int32),
            pltpu.SemaphoreType.DMA,
            pltpu.SemaphoreType.DMA,
            pltpu.SemaphoreType.DMA,
            pltpu.SemaphoreType.DMA,
            pltpu.SemaphoreType.DMA,
            pltpu.SemaphoreType.DMA,
            pltpu.SemaphoreType.DMA,
            pltpu.SemaphoreType.DMA,
            pltpu.VMEM_SHARED((SW + 8, D), jnp.float32),
            pltpu.VMEM_SHARED((CHUNK, D), jnp.float32),
        ],
    )
    return kern(feature, point_idx, pidx2d)


def _tc_body(pa, pb, c0, c1, o):
    tot = c0[0, 0, :] + c1[0, 0, :]
    inv = 1.0 / jnp.maximum(tot, 1.0)
    invc = jnp.reshape(inv, (D, 1))
    o[...] = jax.nn.sigmoid((pa[...] + pb[...]) * invc)


_TC_GRID = (S + D - 1) // D


_tc_finalize = pl.pallas_call(
    _tc_body,
    grid=(_TC_GRID,),
    in_specs=[
        pl.BlockSpec((D, D), lambda i: (i, 0)),
        pl.BlockSpec((D, D), lambda i: (i, 0)),
        pl.BlockSpec((1, 1, D), lambda i: (i, 0, 0)),
        pl.BlockSpec((1, 1, D), lambda i: (i, 0, 0)),
    ],
    out_specs=pl.BlockSpec((D, D), lambda i: (i, 0)),
    out_shape=jax.ShapeDtypeStruct((S, D), jnp.float32),
)


def kernel(feature, point_idx):
    pidx2d = point_idx.reshape(NW, NCHUNK, CHUNK)
    psum, pcnt = _sc_segment_sum(feature, point_idx, pidx2d)
    c0 = pcnt[0].reshape(SP // D, 1, D)
    c1 = pcnt[1].reshape(SP // D, 1, D)
    return _tc_finalize(psum[0], psum[1], c0, c1)
